# Initial kernel scaffold; baseline (speedup 1.0000x reference)
import jax
import jax.numpy as jnp
from jax.experimental import pallas as pl


def kernel(indices, table):
    raise NotImplementedError("write your pallas kernel here")


# speedup vs baseline: 4.0087x; 4.0087x over previous
import functools

import jax
import jax.numpy as jnp
from jax import lax
from jax.experimental import pallas as pl
from jax.experimental.pallas import tpu as pltpu
from jax.experimental.pallas import tpu_sc as plsc

VOCAB = 100000
D = 64
BATCH = 4096
SEQ = 200

NC = 2
NS = 16
NW = NC * NS

TOTAL = BATCH * SEQ
ROWS_PER_W = TOTAL // NW
CHUNK = SEQ
NCHUNK = ROWS_PER_W // CHUNK
LANES = 16
CPR = D // LANES


def _pos_encoding():
    even_i = jnp.arange(0, D, 2).astype(jnp.float32)
    denominator = jnp.power(10000.0, even_i / D)
    position = jnp.arange(SEQ).reshape(SEQ, 1).astype(jnp.float32)
    even_pe = jnp.sin(position / denominator)
    odd_pe = jnp.cos(position / denominator)
    return jnp.stack([even_pe, odd_pe], axis=2).reshape(SEQ, D)


_mesh = plsc.VectorSubcoreMesh(core_axis_name="c", subcore_axis_name="s")


@functools.partial(
    pl.kernel,
    mesh=_mesh,
    compiler_params=pltpu.CompilerParams(use_tc_tiling_on_sc=False),
    out_type=jax.ShapeDtypeStruct((TOTAL, D), jnp.float32),
    scratch_types=[
        pltpu.VMEM((NCHUNK, CHUNK), jnp.int32),
        pltpu.VMEM((SEQ, D), jnp.float32),
        pltpu.VMEM((CHUNK, D), jnp.float32),
        pltpu.VMEM((CHUNK, D), jnp.float32),
        pltpu.SemaphoreType.DMA,
        pltpu.SemaphoreType.DMA,
    ],
)
def _emb_kernel(idx_hbm, table_hbm, pe_hbm, out_hbm,
                idx_v, pe_v, buf0, buf1, sem0, sem1):
    wid = lax.axis_index("s") * NC + lax.axis_index("c")
    row_base = wid * ROWS_PER_W

    pltpu.sync_copy(idx_hbm.at[wid], idx_v)
    pltpu.sync_copy(pe_hbm, pe_v)

    bufs = (buf0, buf1)
    sems = (sem0, sem1)

    pltpu.async_copy(table_hbm.at[idx_v.at[0]], buf0, sem0)
    pltpu.async_copy(table_hbm.at[idx_v.at[1]], buf1, sem1)

    def outer(i, carry):
        h0 = i * 2
        for b in range(2):
            h = h0 + b
            buf = bufs[b]
            sem = sems[b]
            pltpu.make_async_copy(table_hbm.at[idx_v.at[h]], buf, sem).wait()

            def add_row(r, c, _buf=buf):
                for cc in range(CPR):
                    sl = pl.ds(cc * LANES, LANES)
                    _buf[r, sl] = _buf[r, sl] + pe_v[r, sl]
                return c

            lax.fori_loop(0, CHUNK, add_row, 0)

            pltpu.sync_copy(buf, out_hbm.at[pl.ds(row_base + h * CHUNK, CHUNK)])

            @pl.when(h + 2 < NCHUNK)
            def _refill(_buf=buf, _sem=sem, _h=h):
                pltpu.async_copy(table_hbm.at[idx_v.at[_h + 2]], _buf, _sem)

        return carry

    lax.fori_loop(0, NCHUNK // 2, outer, 0)


def kernel(indices, table):
    pe = _pos_encoding()
    idx = indices.reshape(NW, NCHUNK, CHUNK).astype(jnp.int32)
    out = _emb_kernel(idx, table, pe)
    return out.reshape(BATCH, SEQ, D)
